# baseline (speedup 1.0000x reference)
import functools

import jax
import jax.numpy as jnp
from jax import lax
from jax.experimental import pallas as pl
from jax.experimental.pallas import tpu as pltpu
from jax.experimental.pallas import tpu_sc as plsc

VOCAB = 1000000
D_MODEL = 64
D_PAD = 128
SCALE = float(D_MODEL) ** 0.5

B_TOTAL = 4096 * 200
NUM_WORKERS = 32
BPW = B_TOTAL // NUM_WORKERS
GRP = 128
CHUNK = 256
NBUF = 4
NGRP = CHUNK // GRP
NCHUNK = BPW // CHUNK
IDXROWS = BPW // GRP

TBLK = 2048


def _tc_transpose_body(t_ref, o_ref):
    r = lax.broadcasted_iota(jnp.int32, (D_MODEL, D_MODEL), 0)
    c = lax.broadcasted_iota(jnp.int32, (D_MODEL, D_MODEL), 1)
    eye8 = jnp.where(r == c, SCALE, 0.0).astype(jnp.float32)
    o_ref[:, 0:D_MODEL] = lax.dot_general(
        t_ref[...], eye8, (((0,), (0,)), ((), ())),
        preferred_element_type=jnp.float32,
    )


_tc_transpose = pl.pallas_call(
    _tc_transpose_body,
    grid=(pl.cdiv(VOCAB, TBLK),),
    in_specs=[pl.BlockSpec((D_MODEL, TBLK), lambda i: (0, i))],
    out_specs=pl.BlockSpec((TBLK, D_PAD), lambda i: (i, 0)),
    out_shape=jax.ShapeDtypeStruct((VOCAB, D_PAD), jnp.float32),
)


def _make_sc_lookup():
    mesh = plsc.VectorSubcoreMesh(core_axis_name="c", subcore_axis_name="s")

    @functools.partial(
        pl.kernel,
        mesh=mesh,
        compiler_params=pltpu.CompilerParams(use_tc_tiling_on_sc=False),
        out_type=jax.ShapeDtypeStruct((B_TOTAL, D_PAD), jnp.float32),
        scratch_types=[
            pltpu.VMEM((IDXROWS, GRP), jnp.int32),
            pltpu.VMEM((NBUF, CHUNK, D_MODEL), jnp.float32),
        ]
        + [pltpu.SemaphoreType.DMA] * (2 * NBUF),
    )
    def sc_lookup(x_hbm, table_hbm, out_hbm, idx_v, rows_v, *sems):
        gsems = sems[:NBUF]
        wsems = sems[NBUF:]
        wid = lax.axis_index("s") * 2 + lax.axis_index("c")
        rbase0 = wid * BPW

        pltpu.sync_copy(x_hbm.at[pl.ds(wid * IDXROWS, IDXROWS)], idx_v)

        def fire_gather(ci, b):
            for g in range(NGRP):
                pltpu.async_copy(
                    table_hbm.at[idx_v.at[ci * NGRP + g]],
                    rows_v.at[b].at[pl.ds(g * GRP, GRP)],
                    gsems[b],
                )

        def wait_gather(b):
            pltpu.make_async_copy(
                table_hbm.at[pl.ds(0, CHUNK)], rows_v.at[b], gsems[b]
            ).wait()

        def out_dst(ci):
            return out_hbm.at[pl.ds(rbase0 + ci * CHUNK, CHUNK), pl.ds(0, D_MODEL)]

        def fire_write(ci, b):
            pltpu.async_copy(rows_v.at[b], out_dst(ci), wsems[b])

        def wait_write(b):
            pltpu.make_async_copy(rows_v.at[b], out_dst(0), wsems[b]).wait()

        fire_gather(0, 0)
        fire_gather(1, 1)
        for ci in (0, 1):
            wait_gather(ci)
            fire_write(ci, ci)
            fire_gather(ci + 2, ci + 2)

        def quad(p, carry):
            ci0 = 2 + 4 * p
            for slot in range(4):
                b = (2 + slot) % NBUF
                bn = (b + 2) % NBUF
                ci = ci0 + slot
                wait_gather(b)
                fire_write(ci, b)
                wait_write(bn)
                fire_gather(ci + 2, bn)
            return carry

        lax.fori_loop(0, (NCHUNK - 4) // 4, quad, 0)

        for ci in (NCHUNK - 2, NCHUNK - 1):
            b = ci % NBUF
            wait_gather(b)
            fire_write(ci, b)
        for b in range(NBUF):
            wait_write(b)

    return sc_lookup


_sc_lookup = _make_sc_lookup()


def kernel(x, table):
    xg = (x.astype(jnp.int32) * 2).reshape(B_TOTAL // GRP, GRP)
    scaled_pad = _tc_transpose(table.T)
    tbl2 = scaled_pad.reshape(2 * VOCAB, D_MODEL)
    out128 = _sc_lookup(xg, tbl2)
    return out128[:, :D_MODEL].reshape(x.shape[0], x.shape[1], D_MODEL)


# speedup vs baseline: 1.3561x; 1.3561x over previous
import functools

import jax
import jax.numpy as jnp
from jax import lax
from jax.experimental import pallas as pl
from jax.experimental.pallas import tpu as pltpu
from jax.experimental.pallas import tpu_sc as plsc

VOCAB = 1000000
D_MODEL = 64
D_PAD = 128
SCALE = float(D_MODEL) ** 0.5

B_TOTAL = 4096 * 200
NUM_WORKERS = 32
BPW = B_TOTAL // NUM_WORKERS
GRP = 128
CHUNK = 256
NBUF = 4
NGRP = CHUNK // GRP
NCHUNK = BPW // CHUNK
IDXROWS = BPW // GRP

TBLK = 8192


def _tc_transpose_body(t_ref, o_ref):
    o_ref[:, 0:D_MODEL] = t_ref[...].T * SCALE


_tc_transpose = pl.pallas_call(
    _tc_transpose_body,
    grid=(pl.cdiv(VOCAB, TBLK),),
    in_specs=[pl.BlockSpec((D_MODEL, TBLK), lambda i: (0, i))],
    out_specs=pl.BlockSpec((TBLK, D_PAD), lambda i: (i, 0)),
    out_shape=jax.ShapeDtypeStruct((VOCAB, D_PAD), jnp.float32),
)


def _make_sc_lookup():
    mesh = plsc.VectorSubcoreMesh(core_axis_name="c", subcore_axis_name="s")

    @functools.partial(
        pl.kernel,
        mesh=mesh,
        compiler_params=pltpu.CompilerParams(use_tc_tiling_on_sc=False),
        out_type=jax.ShapeDtypeStruct((B_TOTAL, D_PAD), jnp.float32),
        scratch_types=[
            pltpu.VMEM((IDXROWS, GRP), jnp.int32),
            pltpu.VMEM((NBUF, CHUNK, D_MODEL), jnp.float32),
        ]
        + [pltpu.SemaphoreType.DMA] * (2 * NBUF),
    )
    def sc_lookup(x_hbm, table_hbm, out_hbm, idx_v, rows_v, *sems):
        gsems = sems[:NBUF]
        wsems = sems[NBUF:]
        wid = lax.axis_index("s") * 2 + lax.axis_index("c")
        rbase0 = wid * BPW

        pltpu.sync_copy(x_hbm.at[pl.ds(wid * IDXROWS, IDXROWS)], idx_v)

        def fire_gather(ci, b):
            for g in range(NGRP):
                pltpu.async_copy(
                    table_hbm.at[idx_v.at[ci * NGRP + g]],
                    rows_v.at[b].at[pl.ds(g * GRP, GRP)],
                    gsems[b],
                )

        def wait_gather(b):
            pltpu.make_async_copy(
                table_hbm.at[pl.ds(0, CHUNK)], rows_v.at[b], gsems[b]
            ).wait()

        def out_dst(ci):
            return out_hbm.at[pl.ds(rbase0 + ci * CHUNK, CHUNK), pl.ds(0, D_MODEL)]

        def fire_write(ci, b):
            pltpu.async_copy(rows_v.at[b], out_dst(ci), wsems[b])

        def wait_write(b):
            pltpu.make_async_copy(rows_v.at[b], out_dst(0), wsems[b]).wait()

        fire_gather(0, 0)
        fire_gather(1, 1)
        for ci in (0, 1):
            wait_gather(ci)
            fire_write(ci, ci)
            fire_gather(ci + 2, ci + 2)

        def quad(p, carry):
            ci0 = 2 + 4 * p
            for slot in range(4):
                b = (2 + slot) % NBUF
                bn = (b + 2) % NBUF
                ci = ci0 + slot
                wait_gather(b)
                fire_write(ci, b)
                wait_write(bn)
                fire_gather(ci + 2, bn)
            return carry

        lax.fori_loop(0, (NCHUNK - 4) // 4, quad, 0)

        for ci in (NCHUNK - 2, NCHUNK - 1):
            b = ci % NBUF
            wait_gather(b)
            fire_write(ci, b)
        for b in range(NBUF):
            wait_write(b)

    return sc_lookup


_sc_lookup = _make_sc_lookup()


def kernel(x, table):
    xg = (x.astype(jnp.int32) * 2).reshape(B_TOTAL // GRP, GRP)
    scaled_pad = _tc_transpose(table.T)
    tbl2 = scaled_pad.reshape(2 * VOCAB, D_MODEL)
    out128 = _sc_lookup(xg, tbl2)
    return out128[:, :D_MODEL].reshape(x.shape[0], x.shape[1], D_MODEL)


# speedup vs baseline: 1.4058x; 1.0366x over previous
import functools

import jax
import jax.numpy as jnp
from jax import lax
from jax.experimental import pallas as pl
from jax.experimental.pallas import tpu as pltpu
from jax.experimental.pallas import tpu_sc as plsc

VOCAB = 1000000
D_MODEL = 64
D_PAD = 128
SCALE = float(D_MODEL) ** 0.5

B_TOTAL = 4096 * 200
NUM_WORKERS = 32
BPW = B_TOTAL // NUM_WORKERS
GRP = 128
CHUNK = 256
NBUF = 4
NGRP = CHUNK // GRP
NCHUNK = BPW // CHUNK
IDXROWS = BPW // GRP

TBLK = 16384


def _tc_transpose_body(t_ref, o_ref):
    o_ref[:, 0:D_MODEL] = t_ref[...].T * SCALE


_tc_transpose = pl.pallas_call(
    _tc_transpose_body,
    grid=(pl.cdiv(VOCAB, TBLK),),
    in_specs=[pl.BlockSpec((D_MODEL, TBLK), lambda i: (0, i))],
    out_specs=pl.BlockSpec((TBLK, D_PAD), lambda i: (i, 0)),
    out_shape=jax.ShapeDtypeStruct((VOCAB, D_PAD), jnp.float32),
)


def _make_sc_lookup():
    mesh = plsc.VectorSubcoreMesh(core_axis_name="c", subcore_axis_name="s")

    @functools.partial(
        pl.kernel,
        mesh=mesh,
        compiler_params=pltpu.CompilerParams(use_tc_tiling_on_sc=False),
        out_type=jax.ShapeDtypeStruct((B_TOTAL, D_PAD), jnp.float32),
        scratch_types=[
            pltpu.VMEM((IDXROWS, GRP), jnp.int32),
            pltpu.VMEM((NBUF, CHUNK, D_MODEL), jnp.float32),
        ]
        + [pltpu.SemaphoreType.DMA] * (2 * NBUF),
    )
    def sc_lookup(x_hbm, table_hbm, out_hbm, idx_v, rows_v, *sems):
        gsems = sems[:NBUF]
        wsems = sems[NBUF:]
        wid = lax.axis_index("s") * 2 + lax.axis_index("c")
        rbase0 = wid * BPW

        pltpu.sync_copy(x_hbm.at[pl.ds(wid * IDXROWS, IDXROWS)], idx_v)

        def fire_gather(ci, b):
            for g in range(NGRP):
                pltpu.async_copy(
                    table_hbm.at[idx_v.at[ci * NGRP + g]],
                    rows_v.at[b].at[pl.ds(g * GRP, GRP)],
                    gsems[b],
                )

        def wait_gather(b):
            pltpu.make_async_copy(
                table_hbm.at[pl.ds(0, CHUNK)], rows_v.at[b], gsems[b]
            ).wait()

        def out_dst(ci):
            return out_hbm.at[pl.ds(rbase0 + ci * CHUNK, CHUNK), pl.ds(0, D_MODEL)]

        def fire_write(ci, b):
            pltpu.async_copy(rows_v.at[b], out_dst(ci), wsems[b])

        def wait_write(b):
            pltpu.make_async_copy(rows_v.at[b], out_dst(0), wsems[b]).wait()

        fire_gather(0, 0)
        fire_gather(1, 1)
        for ci in (0, 1):
            wait_gather(ci)
            fire_write(ci, ci)
            fire_gather(ci + 2, ci + 2)

        def quad(p, carry):
            ci0 = 2 + 4 * p
            for slot in range(4):
                b = (2 + slot) % NBUF
                bn = (b + 2) % NBUF
                ci = ci0 + slot
                wait_gather(b)
                fire_write(ci, b)
                wait_write(bn)
                fire_gather(ci + 2, bn)
            return carry

        lax.fori_loop(0, (NCHUNK - 4) // 4, quad, 0)

        for ci in (NCHUNK - 2, NCHUNK - 1):
            b = ci % NBUF
            wait_gather(b)
            fire_write(ci, b)
        for b in range(NBUF):
            wait_write(b)

    return sc_lookup


_sc_lookup = _make_sc_lookup()


def kernel(x, table):
    xg = (x.astype(jnp.int32) * 2).reshape(B_TOTAL // GRP, GRP)
    scaled_pad = _tc_transpose(table.T)
    tbl2 = scaled_pad.reshape(2 * VOCAB, D_MODEL)
    out128 = _sc_lookup(xg, tbl2)
    return out128[:, :D_MODEL].reshape(x.shape[0], x.shape[1], D_MODEL)


# speedup vs baseline: 1.4224x; 1.0118x over previous
import functools

import jax
import jax.numpy as jnp
from jax import lax
from jax.experimental import pallas as pl
from jax.experimental.pallas import tpu as pltpu
from jax.experimental.pallas import tpu_sc as plsc

VOCAB = 1000000
D_MODEL = 64
D_PAD = 128
SCALE = float(D_MODEL) ** 0.5

B_TOTAL = 4096 * 200
NUM_WORKERS = 32
BPW = B_TOTAL // NUM_WORKERS
GRP = 128
CHUNK = 256
NBUF = 4
NGRP = CHUNK // GRP
NCHUNK = BPW // CHUNK
IDXROWS = BPW // GRP

TBLK = 32768


def _tc_transpose_body(t_ref, o_ref):
    o_ref[:, 0:D_MODEL] = t_ref[...].T * SCALE


_tc_transpose = pl.pallas_call(
    _tc_transpose_body,
    grid=(pl.cdiv(VOCAB, TBLK),),
    in_specs=[pl.BlockSpec((D_MODEL, TBLK), lambda i: (0, i))],
    out_specs=pl.BlockSpec((TBLK, D_PAD), lambda i: (i, 0)),
    out_shape=jax.ShapeDtypeStruct((VOCAB, D_PAD), jnp.float32),
)


def _make_sc_lookup():
    mesh = plsc.VectorSubcoreMesh(core_axis_name="c", subcore_axis_name="s")

    @functools.partial(
        pl.kernel,
        mesh=mesh,
        compiler_params=pltpu.CompilerParams(use_tc_tiling_on_sc=False),
        out_type=jax.ShapeDtypeStruct((B_TOTAL, D_PAD), jnp.float32),
        scratch_types=[
            pltpu.VMEM((IDXROWS, GRP), jnp.int32),
            pltpu.VMEM((NBUF, CHUNK, D_MODEL), jnp.float32),
        ]
        + [pltpu.SemaphoreType.DMA] * (2 * NBUF),
    )
    def sc_lookup(x_hbm, table_hbm, out_hbm, idx_v, rows_v, *sems):
        gsems = sems[:NBUF]
        wsems = sems[NBUF:]
        wid = lax.axis_index("s") * 2 + lax.axis_index("c")
        rbase0 = wid * BPW

        pltpu.sync_copy(x_hbm.at[pl.ds(wid * IDXROWS, IDXROWS)], idx_v)

        def fire_gather(ci, b):
            for g in range(NGRP):
                pltpu.async_copy(
                    table_hbm.at[idx_v.at[ci * NGRP + g]],
                    rows_v.at[b].at[pl.ds(g * GRP, GRP)],
                    gsems[b],
                )

        def wait_gather(b):
            pltpu.make_async_copy(
                table_hbm.at[pl.ds(0, CHUNK)], rows_v.at[b], gsems[b]
            ).wait()

        def out_dst(ci):
            return out_hbm.at[pl.ds(rbase0 + ci * CHUNK, CHUNK), pl.ds(0, D_MODEL)]

        def fire_write(ci, b):
            pltpu.async_copy(rows_v.at[b], out_dst(ci), wsems[b])

        def wait_write(b):
            pltpu.make_async_copy(rows_v.at[b], out_dst(0), wsems[b]).wait()

        fire_gather(0, 0)
        fire_gather(1, 1)
        for ci in (0, 1):
            wait_gather(ci)
            fire_write(ci, ci)
            fire_gather(ci + 2, ci + 2)

        def quad(p, carry):
            ci0 = 2 + 4 * p
            for slot in range(4):
                b = (2 + slot) % NBUF
                bn = (b + 2) % NBUF
                ci = ci0 + slot
                wait_gather(b)
                fire_write(ci, b)
                wait_write(bn)
                fire_gather(ci + 2, bn)
            return carry

        lax.fori_loop(0, (NCHUNK - 4) // 4, quad, 0)

        for ci in (NCHUNK - 2, NCHUNK - 1):
            b = ci % NBUF
            wait_gather(b)
            fire_write(ci, b)
        for b in range(NBUF):
            wait_write(b)

    return sc_lookup


_sc_lookup = _make_sc_lookup()


def kernel(x, table):
    xg = (x.astype(jnp.int32) * 2).reshape(B_TOTAL // GRP, GRP)
    scaled_pad = _tc_transpose(table.T)
    tbl2 = scaled_pad.reshape(2 * VOCAB, D_MODEL)
    out128 = _sc_lookup(xg, tbl2)
    return out128[:, :D_MODEL].reshape(x.shape[0], x.shape[1], D_MODEL)
